# baseline (speedup 1.0000x reference)
import jax
import jax.numpy as jnp
from jax.experimental import pallas as pl
from jax.experimental.pallas import tpu as pltpu

_EMBED = 768
_NEXP = 8
_CHUNK = 1024
_NBUF = 2


def _router_body(x_hbm, w_ref, g_ref, i_ref, tl_ref, load_ref, xbuf, sems):
    n = x_hbm.shape[0]
    nch = n // _CHUNK

    def start(g):
        b = g % _NBUF
        pltpu.make_async_copy(
            x_hbm.at[pl.ds(g * _CHUNK, _CHUNK)], xbuf.at[b], sems.at[b]
        ).start()

    def wait(g):
        b = g % _NBUF
        pltpu.make_async_copy(
            x_hbm.at[pl.ds(g * _CHUNK, _CHUNK)], xbuf.at[b], sems.at[b]
        ).wait()

    load_ref[:] = jnp.zeros_like(load_ref)
    start(0)
    for c in range(nch):
        if c + 1 < nch:
            start(c + 1)
        wait(c)
        logits = jax.lax.dot_general(
            w_ref[:],
            xbuf[c % _NBUF],
            (((1,), (1,)), ((), ())),
            preferred_element_type=jnp.float32,
        )
        eidx = jax.lax.broadcasted_iota(jnp.int32, logits.shape, 0)
        neg = jnp.float32(-jnp.inf)

        l1 = jnp.max(logits, axis=0, keepdims=True)
        i1 = jnp.min(
            jnp.where(logits == l1, eidx, _NEXP), axis=0, keepdims=True
        )
        masked2 = jnp.where(eidx == i1, neg, logits)
        l2 = jnp.max(masked2, axis=0, keepdims=True)
        i2 = jnp.min(
            jnp.where(masked2 == l2, eidx, _NEXP), axis=0, keepdims=True
        )

        e21 = jnp.exp(l2 - l1)
        denom = 1.0 + e21
        g1 = 1.0 / denom
        g2 = e21 / denom

        sl = pl.ds(c * _CHUNK, _CHUNK)
        g_ref[0:1, sl] = g1
        g_ref[1:2, sl] = g2
        i_ref[0:1, sl] = i1
        i_ref[1:2, sl] = i2
        tl_ref[0:1, sl] = l1
        tl_ref[1:2, sl] = l2

        part = jnp.sum(
            jnp.where(eidx == i1, g1, 0.0) + jnp.where(eidx == i2, g2, 0.0),
            axis=1,
            keepdims=True,
        )
        load_ref[:, 0:1] += part


@jax.jit
def kernel(x, W):
    b, s, d = x.shape
    n = b * s
    x2 = x.reshape(n, d)

    g_t, i_t, tl_t, load = pl.pallas_call(
        _router_body,
        in_specs=[
            pl.BlockSpec(memory_space=pl.ANY),
            pl.BlockSpec((_NEXP, d), lambda: (0, 0)),
        ],
        out_specs=[
            pl.BlockSpec((2, n), lambda: (0, 0)),
            pl.BlockSpec((2, n), lambda: (0, 0)),
            pl.BlockSpec((2, n), lambda: (0, 0)),
            pl.BlockSpec((_NEXP, 128), lambda: (0, 0)),
        ],
        out_shape=[
            jax.ShapeDtypeStruct((2, n), jnp.float32),
            jax.ShapeDtypeStruct((2, n), jnp.int32),
            jax.ShapeDtypeStruct((2, n), jnp.float32),
            jax.ShapeDtypeStruct((_NEXP, 128), jnp.float32),
        ],
        scratch_shapes=[
            pltpu.VMEM((_NBUF, _CHUNK, _EMBED), jnp.float32),
            pltpu.SemaphoreType.DMA((_NBUF,)),
        ],
    )(x2, W)

    return (
        g_t.T.reshape(b, s, 2),
        i_t.T.reshape(b, s, 2),
        load[:, 0],
        tl_t.T.reshape(b, s, 2),
    )


# speedup vs baseline: 1.2315x; 1.2315x over previous
import jax
import jax.numpy as jnp
from jax.experimental import pallas as pl
from jax.experimental.pallas import tpu as pltpu

_EMBED = 768
_NEXP = 8
_CHUNK = 1024
_NBUF = 4


def _router_body(x_hbm, w_ref, g_ref, i_ref, tl_ref, load_ref, xbuf, sems):
    n = x_hbm.shape[0]
    nch = n // _CHUNK

    def start(g):
        b = g % _NBUF
        pltpu.make_async_copy(
            x_hbm.at[pl.ds(g * _CHUNK, _CHUNK)], xbuf.at[b], sems.at[b]
        ).start()

    def wait(g):
        b = g % _NBUF
        pltpu.make_async_copy(
            x_hbm.at[pl.ds(g * _CHUNK, _CHUNK)], xbuf.at[b], sems.at[b]
        ).wait()

    load_ref[:] = jnp.zeros_like(load_ref)
    for g in range(_NBUF - 1):
        start(g)
    for c in range(nch):
        if c + _NBUF - 1 < nch:
            start(c + _NBUF - 1)
        wait(c)
        logits = jax.lax.dot_general(
            w_ref[:],
            xbuf[c % _NBUF],
            (((1,), (1,)), ((), ())),
            preferred_element_type=jnp.float32,
        )
        eidx = jax.lax.broadcasted_iota(jnp.int32, logits.shape, 0)
        neg = jnp.float32(-jnp.inf)

        l1 = jnp.max(logits, axis=0, keepdims=True)
        i1 = jnp.min(
            jnp.where(logits == l1, eidx, _NEXP), axis=0, keepdims=True
        )
        masked2 = jnp.where(eidx == i1, neg, logits)
        l2 = jnp.max(masked2, axis=0, keepdims=True)
        i2 = jnp.min(
            jnp.where(masked2 == l2, eidx, _NEXP), axis=0, keepdims=True
        )

        e21 = jnp.exp(l2 - l1)
        denom = 1.0 + e21
        g1 = 1.0 / denom
        g2 = e21 / denom

        sl = pl.ds(c * _CHUNK, _CHUNK)
        g_ref[0:1, sl] = g1
        g_ref[1:2, sl] = g2
        i_ref[0:1, sl] = i1
        i_ref[1:2, sl] = i2
        tl_ref[0:1, sl] = l1
        tl_ref[1:2, sl] = l2

        part = jnp.sum(
            jnp.where(eidx == i1, g1, 0.0) + jnp.where(eidx == i2, g2, 0.0),
            axis=1,
            keepdims=True,
        )
        load_ref[:, 0:1] += part


@jax.jit
def kernel(x, W):
    b, s, d = x.shape
    n = b * s
    x2 = x.reshape(n, d)

    g_t, i_t, tl_t, load = pl.pallas_call(
        _router_body,
        in_specs=[
            pl.BlockSpec(memory_space=pl.ANY),
            pl.BlockSpec((_NEXP, d), lambda: (0, 0)),
        ],
        out_specs=[
            pl.BlockSpec((2, n), lambda: (0, 0)),
            pl.BlockSpec((2, n), lambda: (0, 0)),
            pl.BlockSpec((2, n), lambda: (0, 0)),
            pl.BlockSpec((_NEXP, 128), lambda: (0, 0)),
        ],
        out_shape=[
            jax.ShapeDtypeStruct((2, n), jnp.float32),
            jax.ShapeDtypeStruct((2, n), jnp.int32),
            jax.ShapeDtypeStruct((2, n), jnp.float32),
            jax.ShapeDtypeStruct((_NEXP, 128), jnp.float32),
        ],
        scratch_shapes=[
            pltpu.VMEM((_NBUF, _CHUNK, _EMBED), jnp.float32),
            pltpu.SemaphoreType.DMA((_NBUF,)),
        ],
    )(x2, W)

    return (
        g_t.T.reshape(b, s, 2),
        i_t.T.reshape(b, s, 2),
        load[:, 0],
        tl_t.T.reshape(b, s, 2),
    )
